# Initial kernel scaffold; baseline (speedup 1.0000x reference)
import functools

import jax
import jax.numpy as jnp
import numpy as np
from jax import lax
from jax.experimental import pallas as pl
from jax.experimental.pallas import tpu as pltpu
from jax.experimental.pallas import tpu_sc as plsc

N = 10000
NB = 16
HIDDEN = 128
OFFSET = 0.0195141
FRACTION = 0.5
PREF = -0.5 * 138.935485 * (1.0 - 1.0 / 78.5)
SA_C = 4.184 * 0.00542 * 100.0

NC, NS, LN = 2, 16, 16
NW = NC * NS
NP = 10240
EP = 163840
EPW = EP // NW
PADN = 10200
KC = 128

_mesh = lambda: plsc.VectorSubcoreMesh(core_axis_name="c", subcore_axis_name="s")


@functools.cache
def make_scalar_gather(F, G):
    nst = EPW // LN
    scratch = ([pltpu.VMEM((NP,), jnp.float32) for _ in range(F)]
               + [pltpu.VMEM((EPW,), jnp.int32)]
               + [pltpu.VMEM((EPW,), jnp.float32) for _ in range(F)])

    @functools.partial(pl.kernel,
                       out_type=jax.ShapeDtypeStruct((G, F, EP), jnp.float32),
                       mesh=_mesh(), scratch_types=scratch)
    def k(tab_hbm, idxs_hbm, out_hbm, *refs):
        tabs, idxv, outs = refs[:F], refs[F], refs[F + 1:]
        wid = lax.axis_index("s") * NC + lax.axis_index("c")
        base = wid * EPW
        for f in range(F):
            pltpu.sync_copy(tab_hbm.at[f], tabs[f])
        for g in range(G):
            pltpu.sync_copy(idxs_hbm.at[g, pl.ds(base, EPW)], idxv)

            def step(i, _):
                iv = idxv[pl.ds(i * LN, LN)]
                for f in range(F):
                    outs[f][pl.ds(i * LN, LN)] = plsc.load_gather(tabs[f], [iv])
                return 0

            lax.fori_loop(0, nst, step, 0)
            for f in range(F):
                pltpu.sync_copy(outs[f], out_hbm.at[g, f, pl.ds(base, EPW)])

    return k


@functools.cache
def make_scalar_scatter(F, T, tmap):
    nst = EPW // LN
    scratch = ([pltpu.VMEM((NP,), jnp.float32) for _ in range(T)]
               + [pltpu.VMEM((EPW,), jnp.int32),
                  pltpu.VMEM((EPW,), jnp.float32)])

    @functools.partial(pl.kernel,
                       out_type=jax.ShapeDtypeStruct((NW, T, NP), jnp.float32),
                       mesh=_mesh(), scratch_types=scratch)
    def k(vals_hbm, idxs_hbm, out_hbm, *refs):
        accs, idxv, valv = refs[:T], refs[T], refs[T + 1]
        wid = lax.axis_index("s") * NC + lax.axis_index("c")
        base = wid * EPW

        def zstep(j, _):
            for t in range(T):
                accs[t][pl.ds(j * LN, LN)] = jnp.zeros((LN,), jnp.float32)
            return 0

        lax.fori_loop(0, NP // LN, zstep, 0)
        for f in range(F):
            pltpu.sync_copy(idxs_hbm.at[f, pl.ds(base, EPW)], idxv)
            pltpu.sync_copy(vals_hbm.at[f, pl.ds(base, EPW)], valv)
            t = tmap[f]

            def step(i, _):
                iv = idxv[pl.ds(i * LN, LN)]
                vv = valv[pl.ds(i * LN, LN)]
                plsc.addupdate_scatter(accs[t], [iv], vv)
                return 0

            lax.fori_loop(0, nst, step, 0)
        for t in range(T):
            pltpu.sync_copy(accs[t], out_hbm.at[wid, t])

    return k


@functools.cache
def make_row_gather(P):
    scratch = [pltpu.VMEM((KC,), jnp.int32),
               pltpu.VMEM((KC, HIDDEN), jnp.float32),
               pltpu.SemaphoreType.DMA]

    @functools.partial(pl.kernel,
                       out_type=jax.ShapeDtypeStruct((P, EP, HIDDEN), jnp.float32),
                       mesh=_mesh(), scratch_types=scratch)
    def k(tabs_hbm, idxs_hbm, out_hbm, ibuf, rbuf, sem):
        wid = lax.axis_index("s") * NC + lax.axis_index("c")
        for p in range(P):
            def chunk(j, _):
                basee = wid * EPW + j * KC
                pltpu.sync_copy(idxs_hbm.at[p, pl.ds(basee, KC)], ibuf)
                pltpu.async_copy(tabs_hbm.at[p].at[ibuf], rbuf, sem).wait()
                pltpu.sync_copy(rbuf, out_hbm.at[p, pl.ds(basee, KC)])
                return 0

            lax.fori_loop(0, EPW // KC, chunk, 0)

    return k


@functools.cache
def make_row_scatter():
    rps = NP // NS
    scratch = [pltpu.VMEM((KC,), jnp.int32),
               pltpu.VMEM((KC, HIDDEN), jnp.float32),
               pltpu.VMEM_SHARED((NP, HIDDEN), jnp.float32),
               pltpu.SemaphoreType.DMA]

    @functools.partial(pl.kernel,
                       out_type=jax.ShapeDtypeStruct((NC, NP, HIDDEN), jnp.float32),
                       mesh=_mesh(), scratch_types=scratch)
    def k(rows_hbm, idx_hbm, z_hbm, out_hbm, ibuf, rbuf, acc, sem):
        cid = lax.axis_index("c")
        sid = lax.axis_index("s")
        wid = sid * NC + cid
        pltpu.sync_copy(z_hbm.at[pl.ds(sid * rps, rps)], acc.at[pl.ds(sid * rps, rps)])
        plsc.subcore_barrier()

        def chunk(j, _):
            basee = wid * EPW + j * KC
            pltpu.sync_copy(idx_hbm.at[pl.ds(basee, KC)], ibuf)
            pltpu.sync_copy(rows_hbm.at[pl.ds(basee, KC)], rbuf)
            pltpu.sync_copy(rbuf, acc.at[ibuf], add=True)
            return 0

        lax.fori_loop(0, EPW // KC, chunk, 0)
        plsc.subcore_barrier()
        pltpu.sync_copy(acc.at[pl.ds(sid * rps, rps)],
                        out_hbm.at[cid, pl.ds(sid * rps, rps)])

    return k


def _padn(a, val=0.0):
    pad = jnp.full((NP - a.shape[0],) + a.shape[1:], val, a.dtype)
    return jnp.concatenate([a, pad], axis=0)


def _pade_idx(a):
    pad = jnp.full((EP - a.shape[0],), PADN, jnp.int32)
    return jnp.concatenate([a.astype(jnp.int32), pad], axis=0)


def _silu(x):
    return x * jax.nn.sigmoid(x)


def _dsilu(x):
    s = jax.nn.sigmoid(x)
    return s * (1.0 + x * (1.0 - s))


def kernel(pos, atoms, batch, edge_index, gnn_edge_index,
           W1_1, b1_1, W2_1, b2_1, W1_2, b1_2, W2_2, b2_2, W1_3, b1_3, W2_3, b2_3):
    posp = _padn(pos)
    atomsp = _padn(atoms, 1.0)
    s = _pade_idx(edge_index[0])
    d = _pade_idx(edge_index[1])
    gs = _pade_idx(gnn_edge_index[0])
    gd = _pade_idx(gnn_edge_index[1])

    q = atomsp[:, 0]
    rho_raw = atomsp[:, 1]
    rho = jnp.clip(rho_raw, 0.05, None)
    scale = 0.5 + 0.5 * atomsp[:, 2]
    sr = scale * rho

    tab6 = jnp.stack([posp[:, 0], posp[:, 1], posp[:, 2], sr, rho, q])

    gat = make_scalar_gather(6, 2)(tab6, jnp.stack([s, d]))
    pxs, pys, pzs, sr_j, qs = gat[0, 0], gat[0, 1], gat[0, 2], gat[0, 3], gat[0, 5]
    pxd, pyd, pzd, rho_i, qd = gat[1, 0], gat[1, 1], gat[1, 2], gat[1, 4], gat[1, 5]
    ggat = make_scalar_gather(6, 2)(tab6, jnp.stack([gs, gd]))
    gdx = ggat[0, 0] - ggat[1, 0]
    gdy = ggat[0, 1] - ggat[1, 1]
    gdz = ggat[0, 2] - ggat[1, 2]
    gde = jnp.sqrt(gdx * gdx + gdy * gdy + gdz * gdz + 1e-12)

    dx, dy, dz = pxs - pxd, pys - pyd, pzs - pzd
    de = jnp.sqrt(dx * dx + dy * dy + dz * dz + 1e-12)
    valid = (s != d).astype(jnp.float32)

    L = jnp.maximum(jnp.abs(de - sr_j), rho_i)
    U = de + sr_j
    A = de - sr_j ** 2 / de
    logLU = jnp.log(L / U)
    Iraw = 0.5 * (1.0 / L - 1.0 / U + 0.25 * A * (1.0 / U ** 2 - 1.0 / L ** 2)
                  + 0.5 / de * logLU)
    gate = valid * (Iraw > 0.0) * (U > rho_i)
    I = jnp.maximum(Iraw, 0.0) * valid * (U > rho_i)
    Lp = (jnp.abs(de - sr_j) > rho_i) * jnp.sign(de - sr_j)
    Ap = 1.0 + sr_j ** 2 / de ** 2
    dIdd = 0.5 * (-Lp / L ** 2 + 1.0 / U ** 2
                  + 0.25 * Ap * (1.0 / U ** 2 - 1.0 / L ** 2)
                  + 0.25 * A * (-2.0 / U ** 3 + 2.0 * Lp / L ** 3)
                  - 0.5 / de ** 2 * logLU
                  + 0.5 / de * (Lp / L - 1.0 / U))

    isum_p = make_scalar_scatter(1, 1, (0,))(I[None], d[None])
    Isum = jnp.sum(isum_p[:, 0, :], axis=0)

    psi = Isum * rho
    uu = psi - 0.8 * psi ** 2 + 4.85 * psi ** 3
    t = jnp.tanh(uu)
    z = 1.0 / rho - t / (rho + OFFSET)
    inv_b = jnp.clip(z, 1e-2, None)
    B = 1.0 / inv_b
    k_node = (-B * B) * (z >= 1e-2) * (-(1.0 - t * t) / (rho + OFFSET)) \
        * (1.0 - 1.6 * psi + 14.55 * psi ** 2) * rho

    x1 = jnp.stack([B, q, rho_raw], axis=1)
    Ws = [(W1_1, b1_1, W2_1, b2_1), (W1_2, b1_2, W2_2, b2_2), (W1_3, b1_3, W2_3, b2_3)]
    zrows = jnp.zeros((NP, HIDDEN), jnp.float32)
    x = x1
    xs, pres, outs = [x1], [], []
    for li, (W1, b1, W2, b2) in enumerate(Ws):
        dxn = x.shape[1]
        W1a, W1b, w1c = W1[:dxn], W1[dxn:2 * dxn], W1[2 * dxn]
        Un = x @ W1a + b1[None, :]
        Vn = x @ W1b
        rows = make_row_gather(2)(jnp.stack([Un, Vn]), jnp.stack([gd, gs]))
        pre = rows[0] + rows[1] + gde[:, None] * w1c[None, :]
        he = _silu(pre)
        aggp = make_row_scatter()(he, gd, zrows)
        agg = aggp[0] + aggp[1]
        if li == 2:
            out = agg @ jnp.pad(W2, ((0, 0), (0, HIDDEN - 2))) \
                + jnp.pad(b2, (0, HIDDEN - 2))[None, :]
        else:
            out = agg @ W2 + b2[None, :]
        pres.append(pre)
        outs.append(out)
        if li < 2:
            x = _silu(out)
            xs.append(x)

    c = outs[2][:, 0]
    sa = outs[2][:, 1]

    radius = rho_raw + OFFSET
    sig_sa = jax.nn.sigmoid(sa)
    sa_en = SA_C * sig_sa * (radius + 0.14) ** 2
    sig_c = jax.nn.sigmoid(c)
    fB = FRACTION + sig_c * (1.0 - FRACTION) * 2.0
    B2 = B * fB

    b2g = make_scalar_gather(1, 2)(B2[None], jnp.stack([s, d]))
    B2s, B2d = b2g[0, 0], b2g[1, 0]
    Bij = B2s * B2d
    X = jnp.exp(-de ** 2 / (4.0 * Bij))
    fgb = jnp.sqrt(de ** 2 + Bij * X)
    epair = PREF * qs * qd / fgb * valid
    g_fgb = -0.5 * PREF * qs * qd * valid / fgb ** 2
    g_de = g_fgb * (2.0 * de - 0.5 * de * X) / (2.0 * fgb)
    g_Bij = g_fgb * X * (1.0 + de ** 2 / (4.0 * Bij)) / (2.0 * fgb)

    sc2 = make_scalar_scatter(3, 2, (0, 1, 1))(
        jnp.stack([epair, g_Bij * B2d, g_Bij * B2s]), jnp.stack([d, s, d]))
    part = jnp.sum(sc2, axis=0)
    eh, g_B2 = part[0], part[1]
    g_B2 = g_B2 - PREF * q * q / B2 ** 2

    gb = 0.5 * eh + PREF * q * q / B2
    energies = gb + sa_en
    batchp = jnp.concatenate([batch.astype(jnp.int32),
                              jnp.full((NP - N,), NB, jnp.int32)])
    energy = jnp.sum(jnp.where(batchp[:, None] == jnp.arange(NB)[None, :],
                               energies[:, None], 0.0), axis=0)[:, None]

    g_B = g_B2 * fB
    g_c = g_B2 * B * sig_c * (1.0 - sig_c)
    g_sa = SA_C * (radius + 0.14) ** 2 * sig_sa * (1.0 - sig_sa)

    g_out = jnp.stack([g_c, g_sa], axis=1)
    g_gde = jnp.zeros((EP,), jnp.float32)
    for li in [2, 1, 0]:
        W1, b1, W2, b2 = Ws[li]
        xin = xs[li]
        dxn = xin.shape[1]
        W1a, W1b, w1c = W1[:dxn], W1[dxn:2 * dxn], W1[2 * dxn]
        g_agg = g_out @ W2.T
        gg = make_row_gather(1)(g_agg[None], gd[None])
        g_pre = gg[0] * _dsilu(pres[li])
        g_gde = g_gde + g_pre @ w1c
        gUp = make_row_scatter()(g_pre, gd, zrows)
        gVp = make_row_scatter()(g_pre, gs, zrows)
        gU = gUp[0] + gUp[1]
        gV = gVp[0] + gVp[1]
        g_x = gU @ W1a.T + gV @ W1b.T
        if li > 0:
            g_out = g_x * _dsilu(outs[li - 1])
        else:
            g_B = g_B + g_x[:, 0]

    g_Isum = g_B * k_node
    gisd = make_scalar_gather(1, 1)(g_Isum[None], d[None])
    g_de = g_de + gisd[0, 0] * gate * dIdd

    cf = g_de / de
    gcf = g_gde / gde
    fx, fy, fz = cf * dx, cf * dy, cf * dz
    gfx, gfy, gfz = gcf * gdx, gcf * gdy, gcf * gdz
    vals = jnp.stack([fx, fy, fz, -fx, -fy, -fz, gfx, gfy, gfz, -gfx, -gfy, -gfz])
    idxs = jnp.stack([s, s, s, d, d, d, gs, gs, gs, gd, gd, gd])
    fpart = make_scalar_scatter(12, 3, (0, 1, 2, 0, 1, 2, 0, 1, 2, 0, 1, 2))(vals, idxs)
    g_pos = jnp.sum(fpart, axis=0)
    forces = -g_pos[:, :N].T
    return energy, forces


# speedup vs baseline: 4.0695x; 4.0695x over previous
import functools

import jax
import jax.numpy as jnp
import numpy as np
from jax import lax
from jax.experimental import pallas as pl
from jax.experimental.pallas import tpu as pltpu
from jax.experimental.pallas import tpu_sc as plsc

N = 10000
NB = 16
HIDDEN = 128
OFFSET = 0.0195141
FRACTION = 0.5
PREF = -0.5 * 138.935485 * (1.0 - 1.0 / 78.5)
SA_C = 4.184 * 0.00542 * 100.0

NC, NS, LN = 2, 16, 16
NW = NC * NS
NP = 10240
EP = 163840
EPW = EP // NW
PADN = 10200
KC = 128

_mesh = lambda: plsc.VectorSubcoreMesh(core_axis_name="c", subcore_axis_name="s")
_cparams = lambda: pltpu.CompilerParams(needs_layout_passes=False)


@functools.cache
def make_scalar_gather(F, G):
    nst = EPW // LN
    scratch = ([pltpu.VMEM((NP,), jnp.float32) for _ in range(F)]
               + [pltpu.VMEM((EPW,), jnp.int32)]
               + [pltpu.VMEM((EPW,), jnp.float32) for _ in range(F)])

    @functools.partial(pl.kernel,
                       out_type=jax.ShapeDtypeStruct((G * F * EP,), jnp.float32),
                       mesh=_mesh(), scratch_types=scratch,
                       compiler_params=_cparams())
    def k(tab_hbm, idxs_hbm, out_hbm, *refs):
        tabs, idxv, outs = refs[:F], refs[F], refs[F + 1:]
        wid = lax.axis_index("s") * NC + lax.axis_index("c")
        base = wid * EPW
        for f in range(F):
            pltpu.sync_copy(tab_hbm.at[pl.ds(f * NP, NP)], tabs[f])
        for g in range(G):
            pltpu.sync_copy(idxs_hbm.at[pl.ds(g * EP + base, EPW)], idxv)

            def step(i, _):
                iv = idxv[pl.ds(i * LN, LN)]
                for f in range(F):
                    outs[f][pl.ds(i * LN, LN)] = plsc.load_gather(tabs[f], [iv])
                return 0

            lax.fori_loop(0, nst, step, 0)
            for f in range(F):
                pltpu.sync_copy(outs[f], out_hbm.at[pl.ds((g * F + f) * EP + base, EPW)])

    return k


@functools.cache
def make_scalar_scatter(F, T, tmap):
    nst = EPW // LN
    scratch = ([pltpu.VMEM((NP,), jnp.float32) for _ in range(T)]
               + [pltpu.VMEM((EPW,), jnp.int32),
                  pltpu.VMEM((EPW,), jnp.float32)])

    @functools.partial(pl.kernel,
                       out_type=jax.ShapeDtypeStruct((NW * T * NP,), jnp.float32),
                       mesh=_mesh(), scratch_types=scratch,
                       compiler_params=_cparams())
    def k(vals_hbm, idxs_hbm, out_hbm, *refs):
        accs, idxv, valv = refs[:T], refs[T], refs[T + 1]
        wid = lax.axis_index("s") * NC + lax.axis_index("c")
        base = wid * EPW

        def zstep(j, _):
            for t in range(T):
                accs[t][pl.ds(j * LN, LN)] = jnp.zeros((LN,), jnp.float32)
            return 0

        lax.fori_loop(0, NP // LN, zstep, 0)
        for f in range(F):
            pltpu.sync_copy(idxs_hbm.at[pl.ds(f * EP + base, EPW)], idxv)
            pltpu.sync_copy(vals_hbm.at[pl.ds(f * EP + base, EPW)], valv)
            t = tmap[f]

            def step(i, _):
                iv = idxv[pl.ds(i * LN, LN)]
                vv = valv[pl.ds(i * LN, LN)]
                plsc.addupdate_scatter(accs[t], [iv], vv)
                return 0

            lax.fori_loop(0, nst, step, 0)
        for t in range(T):
            pltpu.sync_copy(accs[t], out_hbm.at[pl.ds((wid * T + t) * NP, NP)])

    return k


@functools.cache
def make_row_gather(P):
    scratch = [pltpu.VMEM((KC,), jnp.int32),
               pltpu.VMEM((KC, HIDDEN), jnp.float32),
               pltpu.SemaphoreType.DMA]

    @functools.partial(pl.kernel,
                       out_type=jax.ShapeDtypeStruct((P * EP, HIDDEN), jnp.float32),
                       mesh=_mesh(), scratch_types=scratch,
                       compiler_params=_cparams())
    def k(*args):
        tab_hbms = args[:P]
        idxs_hbm, out_hbm, ibuf, rbuf, sem = args[P:]
        wid = lax.axis_index("s") * NC + lax.axis_index("c")
        for p in range(P):
            def chunk(j, _):
                basee = wid * EPW + j * KC
                pltpu.sync_copy(idxs_hbm.at[pl.ds(p * EP + basee, KC)], ibuf)
                pltpu.async_copy(tab_hbms[p].at[ibuf], rbuf, sem).wait()
                pltpu.sync_copy(rbuf, out_hbm.at[pl.ds(p * EP + basee, KC)])
                return 0

            lax.fori_loop(0, EPW // KC, chunk, 0)

    return k


@functools.cache
def make_row_scatter():
    rps = NP // NS
    scratch = [pltpu.VMEM((KC,), jnp.int32),
               pltpu.VMEM((KC, HIDDEN), jnp.float32),
               pltpu.VMEM_SHARED((NP, HIDDEN), jnp.float32),
               pltpu.SemaphoreType.DMA]

    @functools.partial(pl.kernel,
                       out_type=jax.ShapeDtypeStruct((NC * NP, HIDDEN), jnp.float32),
                       mesh=_mesh(), scratch_types=scratch,
                       compiler_params=_cparams())
    def k(rows_hbm, idx_hbm, z_hbm, out_hbm, ibuf, rbuf, acc, sem):
        cid = lax.axis_index("c")
        sid = lax.axis_index("s")
        wid = sid * NC + cid
        pltpu.sync_copy(z_hbm.at[pl.ds(sid * rps, rps)], acc.at[pl.ds(sid * rps, rps)])
        plsc.subcore_barrier()

        def chunk(j, _):
            basee = wid * EPW + j * KC
            pltpu.sync_copy(idx_hbm.at[pl.ds(basee, KC)], ibuf)
            pltpu.sync_copy(rows_hbm.at[pl.ds(basee, KC)], rbuf)
            pltpu.sync_copy(rbuf, acc.at[ibuf], add=True)
            return 0

        lax.fori_loop(0, EPW // KC, chunk, 0)
        plsc.subcore_barrier()
        pltpu.sync_copy(acc.at[pl.ds(sid * rps, rps)],
                        out_hbm.at[pl.ds(cid * NP + sid * rps, rps)])

    return k


def _padn(a, val=0.0):
    pad = jnp.full((NP - a.shape[0],) + a.shape[1:], val, a.dtype)
    return jnp.concatenate([a, pad], axis=0)


def _pade_idx(a):
    pad = jnp.full((EP - a.shape[0],), PADN, jnp.int32)
    return jnp.concatenate([a.astype(jnp.int32), pad], axis=0)


def _silu(x):
    return x * jax.nn.sigmoid(x)


def _dsilu(x):
    s = jax.nn.sigmoid(x)
    return s * (1.0 + x * (1.0 - s))


def kernel(pos, atoms, batch, edge_index, gnn_edge_index,
           W1_1, b1_1, W2_1, b2_1, W1_2, b1_2, W2_2, b2_2, W1_3, b1_3, W2_3, b2_3):
    posp = _padn(pos)
    atomsp = _padn(atoms, 1.0)
    s = _pade_idx(edge_index[0])
    d = _pade_idx(edge_index[1])
    gs = _pade_idx(gnn_edge_index[0])
    gd = _pade_idx(gnn_edge_index[1])

    q = atomsp[:, 0]
    rho_raw = atomsp[:, 1]
    rho = jnp.clip(rho_raw, 0.05, None)
    scale = 0.5 + 0.5 * atomsp[:, 2]
    sr = scale * rho

    tab6 = jnp.concatenate([posp[:, 0], posp[:, 1], posp[:, 2], sr, rho, q])

    gat = make_scalar_gather(6, 2)(tab6, jnp.concatenate([s, d])).reshape(2, 6, EP)
    pxs, pys, pzs, sr_j, qs = gat[0, 0], gat[0, 1], gat[0, 2], gat[0, 3], gat[0, 5]
    pxd, pyd, pzd, rho_i, qd = gat[1, 0], gat[1, 1], gat[1, 2], gat[1, 4], gat[1, 5]
    ggat = make_scalar_gather(6, 2)(tab6, jnp.concatenate([gs, gd])).reshape(2, 6, EP)
    gdx = ggat[0, 0] - ggat[1, 0]
    gdy = ggat[0, 1] - ggat[1, 1]
    gdz = ggat[0, 2] - ggat[1, 2]
    gde = jnp.sqrt(gdx * gdx + gdy * gdy + gdz * gdz + 1e-12)

    dx, dy, dz = pxs - pxd, pys - pyd, pzs - pzd
    de = jnp.sqrt(dx * dx + dy * dy + dz * dz + 1e-12)
    valid = (s != d).astype(jnp.float32)

    L = jnp.maximum(jnp.abs(de - sr_j), rho_i)
    U = de + sr_j
    A = de - sr_j ** 2 / de
    logLU = jnp.log(L / U)
    Iraw = 0.5 * (1.0 / L - 1.0 / U + 0.25 * A * (1.0 / U ** 2 - 1.0 / L ** 2)
                  + 0.5 / de * logLU)
    gate = valid * (Iraw > 0.0) * (U > rho_i)
    I = jnp.maximum(Iraw, 0.0) * valid * (U > rho_i)
    Lp = (jnp.abs(de - sr_j) > rho_i) * jnp.sign(de - sr_j)
    Ap = 1.0 + sr_j ** 2 / de ** 2
    dIdd = 0.5 * (-Lp / L ** 2 + 1.0 / U ** 2
                  + 0.25 * Ap * (1.0 / U ** 2 - 1.0 / L ** 2)
                  + 0.25 * A * (-2.0 / U ** 3 + 2.0 * Lp / L ** 3)
                  - 0.5 / de ** 2 * logLU
                  + 0.5 / de * (Lp / L - 1.0 / U))

    isum_p = make_scalar_scatter(1, 1, (0,))(I, d).reshape(NW, NP)
    Isum = jnp.sum(isum_p, axis=0)

    psi = Isum * rho
    uu = psi - 0.8 * psi ** 2 + 4.85 * psi ** 3
    t = jnp.tanh(uu)
    z = 1.0 / rho - t / (rho + OFFSET)
    inv_b = jnp.clip(z, 1e-2, None)
    B = 1.0 / inv_b
    k_node = (-B * B) * (z >= 1e-2) * (-(1.0 - t * t) / (rho + OFFSET)) \
        * (1.0 - 1.6 * psi + 14.55 * psi ** 2) * rho

    x1 = jnp.stack([B, q, rho_raw], axis=1)
    Ws = [(W1_1, b1_1, W2_1, b2_1), (W1_2, b1_2, W2_2, b2_2), (W1_3, b1_3, W2_3, b2_3)]
    zrows = jnp.zeros((NP, HIDDEN), jnp.float32)
    x = x1
    xs, pres, outs = [x1], [], []
    for li, (W1, b1, W2, b2) in enumerate(Ws):
        dxn = x.shape[1]
        W1a, W1b, w1c = W1[:dxn], W1[dxn:2 * dxn], W1[2 * dxn]
        Un = x @ W1a + b1[None, :]
        Vn = x @ W1b
        rows = make_row_gather(2)(Un, Vn, jnp.concatenate([gd, gs]))
        pre = rows[:EP] + rows[EP:] + gde[:, None] * w1c[None, :]
        he = _silu(pre)
        aggp = make_row_scatter()(he, gd, zrows)
        agg = aggp[:NP] + aggp[NP:]
        if li == 2:
            out = agg @ jnp.pad(W2, ((0, 0), (0, HIDDEN - 2))) \
                + jnp.pad(b2, (0, HIDDEN - 2))[None, :]
        else:
            out = agg @ W2 + b2[None, :]
        pres.append(pre)
        outs.append(out)
        if li < 2:
            x = _silu(out)
            xs.append(x)

    c = outs[2][:, 0]
    sa = outs[2][:, 1]

    radius = rho_raw + OFFSET
    sig_sa = jax.nn.sigmoid(sa)
    sa_en = SA_C * sig_sa * (radius + 0.14) ** 2
    sig_c = jax.nn.sigmoid(c)
    fB = FRACTION + sig_c * (1.0 - FRACTION) * 2.0
    B2 = B * fB

    b2g = make_scalar_gather(1, 2)(B2, jnp.concatenate([s, d]))
    B2s, B2d = b2g[:EP], b2g[EP:]
    Bij = B2s * B2d
    X = jnp.exp(-de ** 2 / (4.0 * Bij))
    fgb = jnp.sqrt(de ** 2 + Bij * X)
    epair = PREF * qs * qd / fgb * valid
    g_fgb = -0.5 * PREF * qs * qd * valid / fgb ** 2
    g_de = g_fgb * (2.0 * de - 0.5 * de * X) / (2.0 * fgb)
    g_Bij = g_fgb * X * (1.0 + de ** 2 / (4.0 * Bij)) / (2.0 * fgb)

    sc2 = make_scalar_scatter(3, 2, (0, 1, 1))(
        jnp.concatenate([epair, g_Bij * B2d, g_Bij * B2s]),
        jnp.concatenate([d, s, d])).reshape(NW, 2, NP)
    part = jnp.sum(sc2, axis=0)
    eh, g_B2 = part[0], part[1]
    g_B2 = g_B2 - PREF * q * q / B2 ** 2

    gb = 0.5 * eh + PREF * q * q / B2
    energies = gb + sa_en
    batchp = jnp.concatenate([batch.astype(jnp.int32),
                              jnp.full((NP - N,), NB, jnp.int32)])
    energy = jnp.sum(jnp.where(batchp[:, None] == jnp.arange(NB)[None, :],
                               energies[:, None], 0.0), axis=0)[:, None]

    g_B = g_B2 * fB
    g_c = g_B2 * B * sig_c * (1.0 - sig_c)
    g_sa = SA_C * (radius + 0.14) ** 2 * sig_sa * (1.0 - sig_sa)

    g_out = jnp.stack([g_c, g_sa], axis=1)
    g_gde = jnp.zeros((EP,), jnp.float32)
    for li in [2, 1, 0]:
        W1, b1, W2, b2 = Ws[li]
        xin = xs[li]
        dxn = xin.shape[1]
        W1a, W1b, w1c = W1[:dxn], W1[dxn:2 * dxn], W1[2 * dxn]
        g_agg = g_out @ W2.T
        gg = make_row_gather(1)(g_agg, gd)
        g_pre = gg * _dsilu(pres[li])
        g_gde = g_gde + g_pre @ w1c
        gUp = make_row_scatter()(g_pre, gd, zrows)
        gVp = make_row_scatter()(g_pre, gs, zrows)
        gU = gUp[:NP] + gUp[NP:]
        gV = gVp[:NP] + gVp[NP:]
        g_x = gU @ W1a.T + gV @ W1b.T
        if li > 0:
            g_out = g_x * _dsilu(outs[li - 1])
        else:
            g_B = g_B + g_x[:, 0]

    g_Isum = g_B * k_node
    gisd = make_scalar_gather(1, 1)(g_Isum, d)
    g_de = g_de + gisd * gate * dIdd

    cf = g_de / de
    gcf = g_gde / gde
    fx, fy, fz = cf * dx, cf * dy, cf * dz
    gfx, gfy, gfz = gcf * gdx, gcf * gdy, gcf * gdz
    vals = jnp.concatenate([fx, fy, fz, -fx, -fy, -fz, gfx, gfy, gfz, -gfx, -gfy, -gfz])
    idxs = jnp.concatenate([s, s, s, d, d, d, gs, gs, gs, gd, gd, gd])
    fpart = make_scalar_scatter(12, 3, (0, 1, 2, 0, 1, 2, 0, 1, 2, 0, 1, 2))(vals, idxs)
    g_pos = jnp.sum(fpart.reshape(NW, 3, NP), axis=0)
    forces = -g_pos[:, :N].T
    return energy, forces
